# baseline (speedup 1.0000x reference)
import functools

import jax
import jax.numpy as jnp
from jax import lax
from jax.experimental import pallas as pl
from jax.experimental.pallas import tpu as pltpu
from jax.experimental.pallas import tpu_sc as plsc

_NC = 2
_NS = 16
_NW = _NC * _NS


@functools.partial(jax.jit, static_argnames=("n_t", "n_b", "n_dr", "n_vt"))
def _sc_embed(wpk, idx_t, *, n_t, n_b, n_dr, n_vt):
    mesh = plsc.VectorSubcoreMesh(core_axis_name="c", subcore_axis_name="s")
    n_br = n_b // 128
    n_ts = min(n_t - 2, 24)

    @functools.partial(
        pl.kernel,
        mesh=mesh,
        compiler_params=pltpu.CompilerParams(
            use_tc_tiling_on_sc=False, needs_layout_passes=False
        ),
        out_type=jax.ShapeDtypeStruct((n_t, n_dr, n_br, 8, 128), jnp.float32),
        scratch_types=[
            pltpu.VMEM((n_vt, 128), jnp.int32),
            pltpu.VMEM((2, n_b), jnp.int32),
            pltpu.VMEM((2, 2, n_br, 128), jnp.float32),
            pltpu.VMEM_SHARED((n_ts, n_b), jnp.int32),
            pltpu.SemaphoreType.DMA,
            pltpu.SemaphoreType.DMA((2,)),
            pltpu.SemaphoreType.DMA((2,)),
            pltpu.SemaphoreType.DMA((2,)),
        ],
    )
    def run(w_hbm, idx_hbm, out_hbm, wcol_v, idx_v, out_v, idx_s,
            wsem, isem, osem0, osem1):
        wid = lax.axis_index("s") * _NC + lax.axis_index("c")
        d0 = wid * 2
        dr = d0 // 8
        dsub = d0 % 8

        def wcol_src():
            return w_hbm.at[pl.ds(wid * n_vt, n_vt)]

        pltpu.async_copy(wcol_src(), wcol_v, wsem)

        @pl.when(lax.axis_index("s") == 0)
        def _():
            pltpu.sync_copy(idx_hbm.at[pl.ds(0, n_ts)], idx_s)

        plsc.subcore_barrier()

        def idx_start_s(t, b):
            pltpu.async_copy(idx_s.at[t], idx_v.at[b], isem.at[b])

        def idx_wait_s(t, b):
            pltpu.make_async_copy(idx_s.at[t], idx_v.at[b], isem.at[b]).wait()

        def idx_start_h(t, b):
            pltpu.async_copy(idx_hbm.at[t], idx_v.at[b], isem.at[b])

        def idx_wait_h(t, b):
            pltpu.make_async_copy(idx_hbm.at[t], idx_v.at[b], isem.at[b]).wait()

        def out_parts(t, b):
            return (
                (out_v.at[b, 0], out_hbm.at[t, dr, :, dsub, :], osem0.at[b]),
                (out_v.at[b, 1], out_hbm.at[t, dr, :, dsub + 1, :], osem1.at[b]),
            )

        def out_start(t, b):
            for src, dst, sem in out_parts(t, b):
                pltpu.async_copy(src, dst, sem)

        def out_wait(t, b):
            for src, dst, sem in out_parts(t, b):
                pltpu.make_async_copy(src, dst, sem).wait()

        idx_start_s(0, 0)
        idx_start_s(1, 1)
        pltpu.make_async_copy(wcol_src(), wcol_v, wsem).wait()

        himask = jnp.int32(-65536)

        def token_work(t, b, idx_wait):
            idx_wait(t, b)

            @pl.when(t >= 2)
            def _():
                out_wait(t - 2, b)

            @plsc.parallel_loop(0, n_b // 16, unroll=8)
            def _gather16(v):
                vec = idx_v[b, pl.ds(v * 16, 16)]
                g = plsc.load_gather(
                    wcol_v, [lax.shift_right_logical(vec, 7), vec & 127]
                )
                lane = pl.ds((v % 8) * 16, 16)
                out_v[b, 0, v // 8, lane] = plsc.bitcast(
                    lax.shift_left(g, 16), jnp.float32
                )
                out_v[b, 1, v // 8, lane] = plsc.bitcast(g & himask, jnp.float32)

            out_start(t, b)

        def token_s(t, carry):
            b = t % 2
            token_work(t, b, idx_wait_s)
            idx_start_s(t + 2, b)
            return carry

        lax.fori_loop(0, n_ts - 2, token_s, 0)
        token_work(n_ts - 2, (n_ts - 2) % 2, idx_wait_s)
        idx_start_h(n_ts, n_ts % 2)
        token_work(n_ts - 1, (n_ts - 1) % 2, idx_wait_s)
        idx_start_h(n_ts + 1, (n_ts + 1) % 2)

        def token_h(t, carry):
            b = t % 2
            token_work(t, b, idx_wait_h)
            idx_start_h(t + 2, b)
            return carry

        lax.fori_loop(n_ts, n_t - 2, token_h, 0)
        token_work(n_t - 2, (n_t - 2) % 2, idx_wait_h)
        token_work(n_t - 1, (n_t - 1) % 2, idx_wait_h)
        out_wait(n_t - 2, n_t % 2)
        out_wait(n_t - 1, (n_t - 1) % 2)

    return run(wpk, idx_t)


def kernel(input_ids, weight):
    n_b, n_t = input_ids.shape
    n_v, n_d = weight.shape
    idx_t = input_ids.T.astype(jnp.int32)
    n_vt = (n_v + 127) // 128
    n_dr = n_d // 8

    a = jnp.pad(weight.astype(jnp.bfloat16), ((0, n_vt * 128 - n_v), (0, 0)))
    u = lax.bitcast_convert_type(a, jnp.uint16).astype(jnp.uint32)
    p = u[:, 0::2] | (u[:, 1::2] << 16)
    wpk = lax.bitcast_convert_type(
        p.T.reshape(_NW * n_vt, 128), jnp.int32
    )

    out5 = _sc_embed(wpk, idx_t, n_t=n_t, n_b=n_b, n_dr=n_dr, n_vt=n_vt)
    return out5.transpose(2, 4, 0, 1, 3).reshape(n_b, n_t, n_d)


# speedup vs baseline: 2.7780x; 2.7780x over previous
import functools

import jax
import jax.numpy as jnp
from jax import lax
from jax.experimental import pallas as pl
from jax.experimental.pallas import tpu as pltpu
from jax.experimental.pallas import tpu_sc as plsc

_NC = 2
_NS = 16
_NW = _NC * _NS


@functools.partial(jax.jit, static_argnames=("n_t", "n_b", "n_dr", "n_vt"))
def _sc_embed(w4, idx_t, *, n_t, n_b, n_dr, n_vt):
    mesh = plsc.VectorSubcoreMesh(core_axis_name="c", subcore_axis_name="s")
    n_br = n_b // 128
    d_per_w = (n_dr * 8) // _NW

    @functools.partial(
        pl.kernel,
        mesh=mesh,
        compiler_params=pltpu.CompilerParams(
            use_tc_tiling_on_sc=False, needs_layout_passes=False
        ),
        out_type=jax.ShapeDtypeStruct((n_t, n_dr, n_br, 8, 128), jnp.float32),
        scratch_types=[
            pltpu.VMEM((n_vt, 128), jnp.float32),
            pltpu.VMEM((2, n_b), jnp.int32),
            pltpu.VMEM((2, n_br, 128), jnp.float32),
            pltpu.VMEM_SHARED((n_t, n_b), jnp.int32),
            pltpu.SemaphoreType.DMA,
            pltpu.SemaphoreType.DMA((2,)),
            pltpu.SemaphoreType.DMA((2,)),
        ],
    )
    def run(w_hbm, idx_hbm, out_hbm, wcol_v, idx_v, out_v, idx_s, wsem, isem, osem):
        wid = lax.axis_index("s") * _NC + lax.axis_index("c")

        @pl.when(lax.axis_index("s") == 0)
        def _():
            pltpu.sync_copy(idx_hbm, idx_s)

        plsc.subcore_barrier()

        for p in range(d_per_w):
            d = wid * d_per_w + p
            dr = d // 8
            dsub = d % 8

            def wcol_src():
                return w_hbm.at[dr, :, dsub, :]

            pltpu.async_copy(wcol_src(), wcol_v, wsem)

            def idx_start(t, b):
                pltpu.async_copy(idx_s.at[t], idx_v.at[b], isem.at[b])

            def idx_wait(t, b):
                pltpu.make_async_copy(
                    idx_s.at[t], idx_v.at[b], isem.at[b]
                ).wait()

            def out_dst(t):
                return out_hbm.at[t, dr, :, dsub, :]

            def out_start(t, b):
                pltpu.async_copy(out_v.at[b], out_dst(t), osem.at[b])

            def out_wait(t, b):
                pltpu.make_async_copy(out_v.at[b], out_dst(t), osem.at[b]).wait()

            idx_start(0, 0)
            idx_start(1, 1)
            pltpu.make_async_copy(wcol_src(), wcol_v, wsem).wait()

            def token_work(t, b):
                idx_wait(t, b)

                @pl.when(t >= 2)
                def _():
                    out_wait(t - 2, b)

                @plsc.parallel_loop(0, n_b // 16, unroll=8)
                def _gather16(v):
                    vec = idx_v[b, pl.ds(v * 16, 16)]
                    vals = plsc.load_gather(
                        wcol_v, [lax.shift_right_logical(vec, 7), vec & 127]
                    )
                    out_v[b, v // 8, pl.ds((v % 8) * 16, 16)] = vals

                out_start(t, b)

            def token(t, carry):
                b = t % 2
                token_work(t, b)
                idx_start(t + 2, b)
                return carry

            lax.fori_loop(0, n_t - 2, token, 0)
            token_work(n_t - 2, (n_t - 2) % 2)
            token_work(n_t - 1, (n_t - 1) % 2)
            out_wait(n_t - 2, n_t % 2)
            out_wait(n_t - 1, (n_t - 1) % 2)

    return run(w4, idx_t)


def kernel(input_ids, weight):
    n_b, n_t = input_ids.shape
    n_v, n_d = weight.shape
    idx_t = input_ids.T.astype(jnp.int32)
    n_vt = (n_v + 127) // 128
    wp = jnp.pad(weight, ((0, n_vt * 128 - n_v), (0, 0)))
    n_dr = n_d // 8
    w4 = wp.T.reshape(n_dr, 8, n_vt, 128).transpose(0, 2, 1, 3)
    out5 = _sc_embed(w4, idx_t, n_t=n_t, n_b=n_b, n_dr=n_dr, n_vt=n_vt)
    return out5.transpose(2, 4, 0, 1, 3).reshape(n_b, n_t, n_d)
